# Initial kernel scaffold; baseline (speedup 1.0000x reference)
import functools

import jax
import jax.numpy as jnp
from jax import lax
from jax.experimental import pallas as pl
from jax.experimental.pallas import tpu as pltpu
from jax.experimental.pallas import tpu_sc as plsc

_NC = 2
_NS = 16
_NW = _NC * _NS


@functools.lru_cache(maxsize=None)
def _make_gather(V, D, B, C):
    b_per_w = B // _NW
    n_chunks = b_per_w // C
    mesh = plsc.VectorSubcoreMesh(
        core_axis_name="c", subcore_axis_name="s", num_cores=_NC, num_subcores=_NS
    )

    @functools.partial(
        pl.kernel,
        out_type=jax.ShapeDtypeStruct((B, D), jnp.float32),
        mesh=mesh,
        scratch_types=[
            pltpu.VMEM((b_per_w,), jnp.int32),
            pltpu.VMEM((C, D), jnp.float32),
            pltpu.SemaphoreType.DMA,
        ],
    )
    def gather_kernel(idx_hbm, table_hbm, out_hbm, idx_v, rows_v, sem):
        wid = lax.axis_index("s") * _NC + lax.axis_index("c")
        base = wid * b_per_w
        pltpu.sync_copy(idx_hbm.at[pl.ds(base, b_per_w)], idx_v)

        def body(j, carry):
            pltpu.async_copy(
                table_hbm.at[idx_v.at[pl.ds(j * C, C)]], rows_v, sem
            ).wait()
            pltpu.sync_copy(rows_v, out_hbm.at[pl.ds(base + j * C, C)])
            return carry

        lax.fori_loop(0, n_chunks, body, 0)

    return gather_kernel


def kernel(inputs, weight):
    B = inputs.shape[0] * inputs.shape[1]
    flat_idx = inputs.reshape(B).astype(jnp.int32)
    out = _make_gather(weight.shape[0], weight.shape[1], B, 1024)(flat_idx, weight)
    return out.reshape(inputs.shape[0], inputs.shape[1], weight.shape[1])


# speedup vs baseline: 1.1024x; 1.1024x over previous
import functools

import jax
import jax.numpy as jnp
from jax import lax
from jax.experimental import pallas as pl
from jax.experimental.pallas import tpu as pltpu
from jax.experimental.pallas import tpu_sc as plsc

_NC = 2
_NS = 16
_NW = _NC * _NS


@functools.lru_cache(maxsize=None)
def _make_gather(V, D, B, C):
    b_per_w = B // _NW
    n_chunks = b_per_w // C
    mesh = plsc.VectorSubcoreMesh(
        core_axis_name="c", subcore_axis_name="s", num_cores=_NC, num_subcores=_NS
    )

    @functools.partial(
        pl.kernel,
        out_type=jax.ShapeDtypeStruct((B, D), jnp.float32),
        mesh=mesh,
        scratch_types=[
            pltpu.VMEM((b_per_w,), jnp.int32),
            pltpu.VMEM((C, D), jnp.float32),
            pltpu.SemaphoreType.DMA,
        ],
        compiler_params=pltpu.CompilerParams(use_tc_tiling_on_sc=False),
    )
    def gather_kernel(idx_hbm, table_hbm, out_hbm, idx_v, rows_v, sem):
        wid = lax.axis_index("s") * _NC + lax.axis_index("c")
        base = wid * b_per_w
        pltpu.sync_copy(idx_hbm.at[pl.ds(base, b_per_w)], idx_v)

        def body(j, carry):
            pltpu.async_copy(
                table_hbm.at[idx_v.at[pl.ds(j * C, C)]], rows_v, sem
            ).wait()
            pltpu.sync_copy(rows_v, out_hbm.at[pl.ds(base + j * C, C)])
            return carry

        lax.fori_loop(0, n_chunks, body, 0)

    return gather_kernel


def kernel(inputs, weight):
    B = inputs.shape[0] * inputs.shape[1]
    flat_idx = inputs.reshape(B).astype(jnp.int32)
    out = _make_gather(weight.shape[0], weight.shape[1], B, 1024)(flat_idx, weight)
    return out.reshape(inputs.shape[0], inputs.shape[1], weight.shape[1])


# speedup vs baseline: 1.1096x; 1.0065x over previous
import functools

import jax
import jax.numpy as jnp
from jax import lax
from jax.experimental import pallas as pl
from jax.experimental.pallas import tpu as pltpu
from jax.experimental.pallas import tpu_sc as plsc

_NC = 2
_NS = 16
_NW = _NC * _NS


_NBUF = 4


@functools.lru_cache(maxsize=None)
def _make_gather(V, D, B, C):
    b_per_w = B // _NW
    n_outer = b_per_w // (C * _NBUF)
    mesh = plsc.VectorSubcoreMesh(
        core_axis_name="c", subcore_axis_name="s", num_cores=_NC, num_subcores=_NS
    )

    @functools.partial(
        pl.kernel,
        out_type=jax.ShapeDtypeStruct((B, D), jnp.float32),
        mesh=mesh,
        scratch_types=[
            pltpu.VMEM((b_per_w,), jnp.int32),
            [pltpu.VMEM((C, D), jnp.float32) for _ in range(_NBUF)],
            [pltpu.SemaphoreType.DMA for _ in range(_NBUF)],
        ],
        compiler_params=pltpu.CompilerParams(use_tc_tiling_on_sc=False),
    )
    def gather_kernel(idx_hbm, table_hbm, out_hbm, idx_v, rows, sems):
        wid = lax.axis_index("s") * _NC + lax.axis_index("c")
        base = wid * b_per_w
        pltpu.sync_copy(idx_hbm.at[pl.ds(base, b_per_w)], idx_v)

        def body(j, carry):
            handles = []
            for b in range(_NBUF):
                c = j * _NBUF + b
                handles.append(
                    pltpu.async_copy(
                        table_hbm.at[idx_v.at[pl.ds(c * C, C)]], rows[b], sems[b]
                    )
                )
            for b in range(_NBUF):
                c = j * _NBUF + b
                handles[b].wait()
                pltpu.sync_copy(rows[b], out_hbm.at[pl.ds(base + c * C, C)])
            return carry

        lax.fori_loop(0, n_outer, body, 0)

    return gather_kernel


def kernel(inputs, weight):
    B = inputs.shape[0] * inputs.shape[1]
    flat_idx = inputs.reshape(B).astype(jnp.int32)
    out = _make_gather(weight.shape[0], weight.shape[1], B, 640)(flat_idx, weight)
    return out.reshape(inputs.shape[0], inputs.shape[1], weight.shape[1])


# speedup vs baseline: 1.5250x; 1.3744x over previous
import functools

import jax
import jax.numpy as jnp
from jax import lax
from jax.experimental import pallas as pl
from jax.experimental.pallas import tpu as pltpu
from jax.experimental.pallas import tpu_sc as plsc

_NC = 2
_NS = 16
_NW = _NC * _NS


@functools.lru_cache(maxsize=None)
def _make_lookup(V, D, BB, T):
    DG = D // 8
    BPW = BB // _NW
    NBT = BPW // 128
    TPW = BPW * T
    mesh = plsc.VectorSubcoreMesh(
        core_axis_name="c", subcore_axis_name="s", num_cores=_NC, num_subcores=_NS
    )

    @functools.partial(
        pl.kernel,
        out_type=jax.ShapeDtypeStruct((BB * T * D,), jnp.float32),
        mesh=mesh,
        scratch_types=[
            pltpu.VMEM((TPW,), jnp.int32),
            pltpu.VMEM((TPW,), jnp.int32),
            pltpu.VMEM((BPW, D), jnp.float32),
            pltpu.VMEM((BPW * D,), jnp.float32),
            pltpu.VMEM((BPW * D,), jnp.float32),
            pltpu.SemaphoreType.DMA,
            pltpu.SemaphoreType.DMA,
            pltpu.SemaphoreType.DMA,
        ],
        compiler_params=pltpu.CompilerParams(
            use_tc_tiling_on_sc=False, needs_layout_passes=False
        ),
    )
    def lookup_kernel(
        idx_hbm, table_hbm, out_hbm, idx_v, idx_t, rows, outa, outb, gsem, wsa, wsb
    ):
        wid = lax.axis_index("s") * _NC + lax.axis_index("c")
        iota = lax.iota(jnp.int32, 16)
        lane0 = jnp.zeros((16,), jnp.int32)

        pltpu.sync_copy(idx_hbm.at[pl.ds(wid * TPW, TPW)], idx_v)

        def tr_t(t, carry):
            def tr_b(blc, carry2):
                src = (blc * 16 + iota) * T + t
                idx_t[pl.ds(t * BPW + blc * 16, 16)] = plsc.load_gather(
                    idx_v, [src]
                )
                return carry2

            lax.fori_loop(0, BPW // 16, tr_b, 0)
            return carry

        lax.fori_loop(0, T, tr_t, 0)

        def woff(t, dg):
            return ((t * DG + dg) * (BB // 128) + NBT * wid) * 1024

        def process(t, outbuf, wsem):
            @pl.when(t >= 2)
            def _():
                for dg in range(DG):
                    pltpu.make_async_copy(
                        outbuf.at[pl.ds(dg * NBT * 1024, NBT * 1024)],
                        out_hbm.at[pl.ds(woff(t, dg), NBT * 1024)],
                        wsem,
                    ).wait()

            pltpu.async_copy(
                table_hbm.at[idx_t.at[pl.ds(t * BPW, BPW)]], rows, gsem
            ).wait()

            def dg_body(dg, carry):
                cols = [lane0 + (dg * 8 + s) for s in range(8)]

                def bt_body(bt, carry2):
                    rids = [bt * 128 + lc * 16 + iota for lc in range(8)]
                    for s in range(8):
                        for lc in range(8):
                            v = plsc.load_gather(rows, [rids[lc], cols[s]])
                            outbuf[
                                pl.ds(((dg * NBT + bt) * 8 + s) * 128 + lc * 16, 16)
                            ] = v
                    return carry2

                lax.fori_loop(0, NBT, bt_body, 0)
                return carry

            lax.fori_loop(0, DG, dg_body, 0)

            for dg in range(DG):
                pltpu.async_copy(
                    outbuf.at[pl.ds(dg * NBT * 1024, NBT * 1024)],
                    out_hbm.at[pl.ds(woff(t, dg), NBT * 1024)],
                    wsem,
                )

        def step(j, carry):
            process(2 * j, outa, wsa)
            process(2 * j + 1, outb, wsb)
            return carry

        lax.fori_loop(0, T // 2, step, 0)

        for dg in range(DG):
            pltpu.make_async_copy(
                outa.at[pl.ds(dg * NBT * 1024, NBT * 1024)],
                out_hbm.at[pl.ds(woff(T - 2, dg), NBT * 1024)],
                wsa,
            ).wait()
            pltpu.make_async_copy(
                outb.at[pl.ds(dg * NBT * 1024, NBT * 1024)],
                out_hbm.at[pl.ds(woff(T - 1, dg), NBT * 1024)],
                wsb,
            ).wait()

    return lookup_kernel


def kernel(inputs, weight):
    BB, T = inputs.shape
    V, D = weight.shape
    flat_idx = inputs.reshape(BB * T).astype(jnp.int32)
    out1d = _make_lookup(V, D, BB, T)(flat_idx, weight)
    s5 = out1d.reshape(T, D // 8, BB // 128, 8, 128)
    return s5.transpose(2, 4, 0, 1, 3).reshape(BB, T, D)


# speedup vs baseline: 1.8273x; 1.1982x over previous
import functools

import jax
import jax.numpy as jnp
from jax import lax
from jax.experimental import pallas as pl
from jax.experimental.pallas import tpu as pltpu
from jax.experimental.pallas import tpu_sc as plsc

_NC = 2
_NS = 16
_NW = _NC * _NS


@functools.lru_cache(maxsize=None)
def _make_lookup(V, D, BB, T):
    DG = D // 8
    BPW = BB // _NW
    NBT = BPW // 128
    TPW = BPW * T
    mesh = plsc.VectorSubcoreMesh(
        core_axis_name="c", subcore_axis_name="s", num_cores=_NC, num_subcores=_NS
    )

    @functools.partial(
        pl.kernel,
        out_type=jax.ShapeDtypeStruct((BB * T * D,), jnp.float32),
        mesh=mesh,
        scratch_types=[
            pltpu.VMEM((TPW,), jnp.int32),
            pltpu.VMEM((TPW,), jnp.int32),
            [pltpu.VMEM((BPW, D), jnp.float32) for _ in range(2)],
            [pltpu.VMEM((BPW * D,), jnp.float32) for _ in range(2)],
            [pltpu.SemaphoreType.DMA for _ in range(2)],
            [pltpu.SemaphoreType.DMA for _ in range(2)],
        ],
        compiler_params=pltpu.CompilerParams(
            use_tc_tiling_on_sc=False, needs_layout_passes=False
        ),
    )
    def lookup_kernel(
        idx_hbm, table_hbm, out_hbm, idx_v, idx_t, rows, outs, gsems, wsems
    ):
        wid = lax.axis_index("s") * _NC + lax.axis_index("c")
        iota = lax.iota(jnp.int32, 16)
        lane0 = jnp.zeros((16,), jnp.int32)

        pltpu.sync_copy(idx_hbm.at[pl.ds(wid * TPW, TPW)], idx_v)

        def tr_t(t, carry):
            def tr_b(blc, carry2):
                src = (blc * 16 + iota) * T + t
                idx_t[pl.ds(t * BPW + blc * 16, 16)] = plsc.load_gather(
                    idx_v, [src]
                )
                return carry2

            lax.fori_loop(0, BPW // 16, tr_b, 0)
            return carry

        lax.fori_loop(0, T, tr_t, 0)

        def woff(t, dg):
            return ((t * DG + dg) * (BB // 128) + NBT * wid) * 1024

        cols = [[lane0 + (dg * 8 + s) for s in range(8)] for dg in range(DG)]

        def start_gather(t, p):
            pltpu.async_copy(
                table_hbm.at[idx_t.at[pl.ds(t * BPW, BPW)]], rows[p], gsems[p]
            )

        def process(t, p):
            outbuf, wsem = outs[p], wsems[p]

            @pl.when(t >= 2)
            def _():
                for dg in range(DG):
                    pltpu.make_async_copy(
                        outbuf.at[pl.ds(dg * NBT * 1024, NBT * 1024)],
                        out_hbm.at[pl.ds(woff(t, dg), NBT * 1024)],
                        wsem,
                    ).wait()

            pltpu.make_async_copy(
                table_hbm.at[idx_t.at[pl.ds(t * BPW, BPW)]], rows[p], gsems[p]
            ).wait()

            def bt_body(bt, carry2):
                rids = [bt * 128 + lc * 16 + iota for lc in range(8)]
                for dg in range(DG):
                    tile0 = ((dg * NBT + bt) * 8) * 128
                    for s in range(8):
                        vs = [
                            plsc.load_gather(rows[p], [rids[lc], cols[dg][s]])
                            for lc in range(8)
                        ]
                        for lc in range(8):
                            outbuf[pl.ds(tile0 + s * 128 + lc * 16, 16)] = vs[lc]
                return carry2

            lax.fori_loop(0, NBT, bt_body, 0)

            for dg in range(DG):
                pltpu.async_copy(
                    outbuf.at[pl.ds(dg * NBT * 1024, NBT * 1024)],
                    out_hbm.at[pl.ds(woff(t, dg), NBT * 1024)],
                    wsem,
                )

            @pl.when(t + 2 < T)
            def _():
                start_gather(t + 2, p)

        start_gather(0, 0)
        start_gather(1, 1)

        def step(j, carry):
            process(2 * j, 0)
            process(2 * j + 1, 1)
            return carry

        lax.fori_loop(0, T // 2, step, 0)

        for dg in range(DG):
            pltpu.make_async_copy(
                outs[0].at[pl.ds(dg * NBT * 1024, NBT * 1024)],
                out_hbm.at[pl.ds(woff(T - 2, dg), NBT * 1024)],
                wsems[0],
            ).wait()
            pltpu.make_async_copy(
                outs[1].at[pl.ds(dg * NBT * 1024, NBT * 1024)],
                out_hbm.at[pl.ds(woff(T - 1, dg), NBT * 1024)],
                wsems[1],
            ).wait()

    return lookup_kernel


def kernel(inputs, weight):
    BB, T = inputs.shape
    V, D = weight.shape
    flat_idx = inputs.reshape(BB * T).astype(jnp.int32)
    out1d = _make_lookup(V, D, BB, T)(flat_idx, weight)
    s5 = out1d.reshape(T, D // 8, BB // 128, 8, 128)
    return s5.transpose(2, 4, 0, 1, 3).reshape(BB, T, D)


# speedup vs baseline: 1.8439x; 1.0091x over previous
import functools

import jax
import jax.numpy as jnp
from jax import lax
from jax.experimental import pallas as pl
from jax.experimental.pallas import tpu as pltpu
from jax.experimental.pallas import tpu_sc as plsc

_NC = 2
_NS = 16
_NW = _NC * _NS


@functools.lru_cache(maxsize=None)
def _make_lookup(V, D, BB, T):
    DG = D // 8
    BPW = BB // _NW
    NBT = BPW // 128
    TPW = BPW * T
    mesh = plsc.VectorSubcoreMesh(
        core_axis_name="c", subcore_axis_name="s", num_cores=_NC, num_subcores=_NS
    )

    @functools.partial(
        pl.kernel,
        out_type=jax.ShapeDtypeStruct((BB * T * D,), jnp.float32),
        mesh=mesh,
        scratch_types=[
            pltpu.VMEM((T, BPW), jnp.int32),
            [pltpu.VMEM((BPW, D), jnp.float32) for _ in range(2)],
            [pltpu.VMEM((BPW * D,), jnp.float32) for _ in range(2)],
            [pltpu.SemaphoreType.DMA for _ in range(2)],
            [pltpu.SemaphoreType.DMA for _ in range(2)],
        ],
        compiler_params=pltpu.CompilerParams(
            use_tc_tiling_on_sc=False, needs_layout_passes=False
        ),
    )
    def lookup_kernel(
        idx_hbm, table_hbm, out_hbm, idx_t, rows, outs, gsems, wsems
    ):
        wid = lax.axis_index("s") * _NC + lax.axis_index("c")
        iota = lax.iota(jnp.int32, 16)
        lane0 = jnp.zeros((16,), jnp.int32)

        pltpu.sync_copy(idx_hbm.at[:, pl.ds(wid * BPW, BPW)], idx_t)

        def woff(t, dg):
            return ((t * DG + dg) * (BB // 128) + NBT * wid) * 1024

        cols = [[lane0 + (dg * 8 + s) for s in range(8)] for dg in range(DG)]

        def start_gather(t, p):
            pltpu.async_copy(
                table_hbm.at[idx_t.at[t]], rows[p], gsems[p]
            )

        def process(t, p):
            outbuf, wsem = outs[p], wsems[p]

            @pl.when(t >= 2)
            def _():
                for dg in range(DG):
                    pltpu.make_async_copy(
                        outbuf.at[pl.ds(dg * NBT * 1024, NBT * 1024)],
                        out_hbm.at[pl.ds(woff(t, dg), NBT * 1024)],
                        wsem,
                    ).wait()

            pltpu.make_async_copy(
                table_hbm.at[idx_t.at[t]], rows[p], gsems[p]
            ).wait()

            def bt_body(bt, carry2):
                rids = [bt * 128 + lc * 16 + iota for lc in range(8)]
                for dg in range(DG):
                    tile0 = ((dg * NBT + bt) * 8) * 128
                    for s in range(8):
                        vs = [
                            plsc.load_gather(rows[p], [rids[lc], cols[dg][s]])
                            for lc in range(8)
                        ]
                        for lc in range(8):
                            outbuf[pl.ds(tile0 + s * 128 + lc * 16, 16)] = vs[lc]
                return carry2

            lax.fori_loop(0, NBT, bt_body, 0)

            for dg in range(DG):
                pltpu.async_copy(
                    outbuf.at[pl.ds(dg * NBT * 1024, NBT * 1024)],
                    out_hbm.at[pl.ds(woff(t, dg), NBT * 1024)],
                    wsem,
                )

            @pl.when(t + 2 < T)
            def _():
                start_gather(t + 2, p)

        start_gather(0, 0)
        start_gather(1, 1)

        def step(j, carry):
            process(2 * j, 0)
            process(2 * j + 1, 1)
            return carry

        lax.fori_loop(0, T // 2, step, 0)

        for dg in range(DG):
            pltpu.make_async_copy(
                outs[0].at[pl.ds(dg * NBT * 1024, NBT * 1024)],
                out_hbm.at[pl.ds(woff(T - 2, dg), NBT * 1024)],
                wsems[0],
            ).wait()
            pltpu.make_async_copy(
                outs[1].at[pl.ds(dg * NBT * 1024, NBT * 1024)],
                out_hbm.at[pl.ds(woff(T - 1, dg), NBT * 1024)],
                wsems[1],
            ).wait()

    return lookup_kernel


def kernel(inputs, weight):
    BB, T = inputs.shape
    V, D = weight.shape
    idx_tm = inputs.T.astype(jnp.int32)
    out1d = _make_lookup(V, D, BB, T)(idx_tm, weight)
    s5 = out1d.reshape(T, D // 8, BB // 128, 8, 128)
    return s5.transpose(2, 4, 0, 1, 3).reshape(BB, T, D)
